# baseline (speedup 1.0000x reference)
import functools

import jax
import jax.numpy as jnp
from jax import lax
from jax.experimental import pallas as pl
from jax.experimental.pallas import tpu as pltpu
from jax.experimental.pallas import tpu_sc as plsc

N = 10000
E = 160000
D = 256
BR = 1000

NSC = 2
NT = 16
NW = NSC * NT
RPW = 320
NPAD = NW * RPW
TRASH = RPW
ACC_R = RPW + 1

SUPER = 16000
NSUP = E // SUPER
OSZ = SUPER + 8
STRIDE = NSUP * OSZ + 64
CLEN = NW * 16

RND = 4096
CH = 32
DEG_W = 16

_sc_mesh = plsc.VectorSubcoreMesh(core_axis_name="c", subcore_axis_name="s")


@functools.partial(
    pl.kernel,
    out_type=[
        jax.ShapeDtypeStruct((NW * STRIDE + RND,), jnp.int32),
        jax.ShapeDtypeStruct((NW * STRIDE + RND,), jnp.int32),
        jax.ShapeDtypeStruct((CLEN,), jnp.int32),
        jax.ShapeDtypeStruct((NPAD * DEG_W,), jnp.float32),
    ],
    mesh=_sc_mesh,
    compiler_params=pltpu.CompilerParams(needs_layout_passes=False),
    scratch_types=[
        pltpu.VMEM((SUPER,), jnp.int32),
        pltpu.VMEM((SUPER,), jnp.int32),
        pltpu.VMEM((SUPER,), jnp.int32),
        pltpu.VMEM((SUPER,), jnp.int32),
        pltpu.VMEM((OSZ,), jnp.int32),
        pltpu.VMEM((OSZ,), jnp.int32),
        pltpu.VMEM((ACC_R * DEG_W,), jnp.float32),
        pltpu.VMEM((64,), jnp.int32),
        pltpu.VMEM((64,), jnp.int32),
        pltpu.VMEM((16,), jnp.int32),
        pltpu.SemaphoreType.DMA,
        pltpu.SemaphoreType.DMA,
    ],
)
def _sc_prep(src_hbm, dst_hbm, csrc_hbm, cloc_hbm, cnt_hbm, deg_hbm,
             sA, dA, sB, dB, osrc, oloc, degacc, safes, safel, cbuf,
             semA, semB):
    sc = lax.axis_index("c")
    t = lax.axis_index("s")
    w = sc * NT + t
    wlo = w * RPW
    obase = pl.multiple_of(w * STRIDE, 8)
    iota16 = lax.iota(jnp.int32, 16)
    ones16 = jnp.ones((16,), jnp.float32)

    zero16 = jnp.zeros((16,), jnp.float32)

    def zbody(r, c):
        degacc[pl.ds(r * 16, 16)] = zero16
        return c
    lax.fori_loop(0, ACC_R * DEG_W // 16, zbody, 0)

    for j in range(4):
        safes[pl.ds(j * 16, 16)] = iota16
        safel[pl.ds(j * 16, 16)] = jnp.full((16,), TRASH, jnp.int32)

    sets = ((sA, dA, semA), (sB, dB, semB))

    def issue(sp, st):
        s_v, d_v, sem = st
        base = sp * SUPER
        pltpu.async_copy(src_hbm.at[pl.ds(base, SUPER)], s_v, sem)
        pltpu.async_copy(dst_hbm.at[pl.ds(base, SUPER)], d_v, sem)

    def drain(sp, st):
        s_v, d_v, sem = st
        base = sp * SUPER
        pltpu.make_async_copy(src_hbm.at[pl.ds(base, SUPER)], s_v, sem).wait()
        pltpu.make_async_copy(dst_hbm.at[pl.ds(base, SUPER)], d_v, sem).wait()

    issue(0, sets[0])
    done = jnp.int32(0)
    for sp in range(NSUP):
        cur = sets[sp % 2]
        drain(sp, cur)
        if sp + 1 < NSUP:
            issue(sp + 1, sets[(sp + 1) % 2])
        s_v, d_v, _ = cur

        def vbody(v, tail):
            base = v * 16
            s16 = s_v[pl.ds(base, 16)]
            d16 = d_v[pl.ds(base, 16)]
            loc = d16 - wlo
            owned = (loc >= 0) & (loc < RPW)
            mi = jnp.where(owned, 1, 0).astype(jnp.int32)
            cs = plsc.cumsum(mi)
            idx = tail + cs - 1
            plsc.store_scatter(osrc, [idx], s16, mask=owned)
            plsc.store_scatter(oloc, [idx], loc, mask=owned)
            clamped = jnp.where(owned, loc, TRASH)
            plsc.addupdate_scatter(degacc, [clamped * DEG_W + iota16], ones16)
            return tail + jnp.sum(mi)

        tail = lax.fori_loop(0, SUPER // 16, vbody, jnp.int32(0))
        padl = (-tail) % 8
        pidx = tail + iota16
        pmask = iota16 < padl
        plsc.store_scatter(osrc, [pidx], iota16, mask=pmask)
        plsc.store_scatter(oloc, [pidx], jnp.full((16,), TRASH, jnp.int32),
                           mask=pmask)
        tail = tail + padl
        off = pl.multiple_of(obase + done, 8)
        pltpu.sync_copy(osrc, csrc_hbm.at[pl.ds(off, OSZ)])
        pltpu.sync_copy(oloc, cloc_hbm.at[pl.ds(off, OSZ)])
        done = done + tail

    off = pl.multiple_of(obase + done, 8)
    pltpu.sync_copy(safes, csrc_hbm.at[pl.ds(off, 64)])
    pltpu.sync_copy(safel, cloc_hbm.at[pl.ds(off, 64)])

    cbuf[...] = jnp.broadcast_to(done, (16,)).astype(jnp.int32)
    pltpu.sync_copy(cbuf, cnt_hbm.at[pl.ds(w * 16, 16)])
    pltpu.sync_copy(degacc.at[pl.ds(0, RPW * DEG_W)],
                    deg_hbm.at[pl.ds(wlo * DEG_W, RPW * DEG_W)])


def _bcast_lane(vec16, lane):
    idx = jnp.broadcast_to(lane, (16, 1)).astype(jnp.int32)
    return lax.gather(
        vec16, idx,
        lax.GatherDimensionNumbers(offset_dims=(), collapsed_slice_dims=(0,),
                                   start_index_map=(0,)),
        slice_sizes=(1,),
        mode=lax.GatherScatterMode.PROMISE_IN_BOUNDS)


def _make_aggregate():
    colj = None

    @functools.partial(
        pl.kernel,
        out_type=jax.ShapeDtypeStruct((NPAD * D,), jnp.float32),
        mesh=_sc_mesh,
        compiler_params=pltpu.CompilerParams(needs_layout_passes=False),
        scratch_types=[
            pltpu.VMEM((RND,), jnp.int32),
            pltpu.VMEM((RND,), jnp.int32),
            pltpu.VMEM((CH, D), jnp.float32),
            pltpu.VMEM((CH, D), jnp.float32),
            pltpu.VMEM((ACC_R * D,), jnp.float32),
            pltpu.VMEM((16,), jnp.int32),
            pltpu.SemaphoreType.DMA,
            pltpu.SemaphoreType.DMA,
        ],
    )
    def agg(g_hbm, csrc_hbm, cloc_hbm, cnt_hbm, s_hbm,
            bsrc, bloc, rowA, rowB, acc, cbuf, semA, semB):
        sc = lax.axis_index("c")
        t = lax.axis_index("s")
        w = sc * NT + t
        wlo = w * RPW
        obase = pl.multiple_of(w * STRIDE, 8)
        iota16 = lax.iota(jnp.int32, 16)
        cols = [jnp.int32(j * 16) + iota16 for j in range(D // 16)]
        zero16 = jnp.zeros((16,), jnp.float32)

        def zbody(r, c):
            acc[pl.ds(r * 16, 16)] = zero16
            return c

        lax.fori_loop(0, ACC_R * D // 16, zbody, 0)
        pltpu.sync_copy(cnt_hbm.at[pl.ds(w * 16, 16)], cbuf)
        cnt = jnp.max(cbuf[...])
        nrnd = (cnt + RND - 1) // RND

        def round_body(r, carry):
            rbase = r * RND
            off = pl.multiple_of(obase + rbase, 8)
            pltpu.sync_copy(csrc_hbm.at[pl.ds(off, RND)], bsrc)
            pltpu.sync_copy(cloc_hbm.at[pl.ds(off, RND)], bloc)
            rcnt = jnp.minimum(jnp.int32(RND), cnt - rbase)
            nch = (rcnt + CH - 1) // CH

            def gidx(ch):
                return bsrc.at[pl.ds(ch * CH, CH)]

            def step(ch, rbuf, sem, nbuf, nsem):
                pltpu.make_async_copy(g_hbm.at[gidx(ch)], rbuf, sem).wait()

                @pl.when(ch + 1 < nch)
                def _():
                    pltpu.async_copy(g_hbm.at[gidx(ch + 1)], nbuf, nsem)

                for grp in range(CH // 16):
                    locv = bloc[pl.ds(ch * CH + grp * 16, 16)] * D
                    for l in range(16):
                        locb = jnp.broadcast_to(locv[l], (16,))
                        e = grp * 16 + l
                        for j in range(D // 16):
                            val = rbuf[e, pl.ds(j * 16, 16)]
                            plsc.addupdate_scatter(acc, [locb + cols[j]], val)

            @pl.when(nch > 0)
            def _():
                pltpu.async_copy(g_hbm.at[gidx(0)], rowA, semA)

            def chunk_body(ch, c):
                even = (ch % 2) == 0

                @pl.when(even)
                def _():
                    step(ch, rowA, semA, rowB, semB)

                @pl.when(jnp.logical_not(even))
                def _():
                    step(ch, rowB, semB, rowA, semA)

                return c

            lax.fori_loop(0, nch, chunk_body, 0)
            return carry

        lax.fori_loop(0, nrnd, round_body, 0)
        out_base = pl.multiple_of(wlo * D, 8)
        pltpu.sync_copy(acc.at[pl.ds(0, RPW * D)],
                        s_hbm.at[pl.ds(out_base, RPW * D)])

    return agg


_sc_aggregate = _make_aggregate()


def _dinv_body(deg_ref, o_ref):
    deg = 1.0 + jnp.sum(deg_ref[...], axis=1, keepdims=True)
    o_ref[...] = lax.rsqrt(deg)


def _dinv(deg16):
    return pl.pallas_call(
        _dinv_body,
        grid=(N // BR,),
        in_specs=[pl.BlockSpec((BR, DEG_W), lambda i: (i, 0))],
        out_specs=pl.BlockSpec((BR, 1), lambda i: (i, 0)),
        out_shape=jax.ShapeDtypeStruct((N, 1), jnp.float32),
    )(deg16)


def _mm_body(x_ref, w_ref, dinv_ref, c_ref, o_ref):
    o_ref[...] = dinv_ref[...] * (jnp.dot(
        x_ref[...], w_ref[...], preferred_element_type=jnp.float32)
        + c_ref[...])


def _mm(x, w, dinv2d, c):
    grid = (x.shape[0] // BR,)
    return pl.pallas_call(
        _mm_body,
        grid=grid,
        in_specs=[
            pl.BlockSpec((BR, x.shape[1]), lambda i: (i, 0)),
            pl.BlockSpec((x.shape[1], D), lambda i: (0, 0)),
            pl.BlockSpec((BR, 1), lambda i: (i, 0)),
            pl.BlockSpec((1, D), lambda i: (0, 0)),
        ],
        out_specs=pl.BlockSpec((BR, D), lambda i: (i, 0)),
        out_shape=jax.ShapeDtypeStruct((x.shape[0], D), jnp.float32),
    )(x, w, dinv2d, c)


def _combine_body(s_ref, g_ref, dinv_ref, b_ref, y_ref, s1_ref, s2_ref):
    i = pl.program_id(0)
    y = dinv_ref[...] * (s_ref[...] + g_ref[...]) + b_ref[...]
    y_ref[...] = y

    @pl.when(i == 0)
    def _init():
        s1_ref[...] = jnp.zeros_like(s1_ref)
        s2_ref[...] = jnp.zeros_like(s2_ref)

    s1_ref[...] += jnp.sum(y, axis=0, keepdims=True)
    s2_ref[...] += jnp.sum(y * y, axis=0, keepdims=True)


def _combine(s_pad, g, dinv2d, b):
    grid = (N // BR,)
    return pl.pallas_call(
        _combine_body,
        grid=grid,
        in_specs=[
            pl.BlockSpec((BR, D), lambda i: (i, 0)),
            pl.BlockSpec((BR, D), lambda i: (i, 0)),
            pl.BlockSpec((BR, 1), lambda i: (i, 0)),
            pl.BlockSpec((1, D), lambda i: (0, 0)),
        ],
        out_specs=[
            pl.BlockSpec((BR, D), lambda i: (i, 0)),
            pl.BlockSpec((1, D), lambda i: (0, 0)),
            pl.BlockSpec((1, D), lambda i: (0, 0)),
        ],
        out_shape=[
            jax.ShapeDtypeStruct((N, D), jnp.float32),
            jax.ShapeDtypeStruct((1, D), jnp.float32),
            jax.ShapeDtypeStruct((1, D), jnp.float32),
        ],
    )(s_pad, g, dinv2d, b)


def _norm_body(y_ref, al_ref, be_ref, o_ref):
    o_ref[...] = y_ref[...] * al_ref[...] + be_ref[...]


def _norm(y, alpha, beta):
    return pl.pallas_call(
        _norm_body,
        grid=(N // BR,),
        in_specs=[
            pl.BlockSpec((BR, D), lambda i: (i, 0)),
            pl.BlockSpec((1, D), lambda i: (0, 0)),
            pl.BlockSpec((1, D), lambda i: (0, 0)),
        ],
        out_specs=pl.BlockSpec((BR, D), lambda i: (i, 0)),
        out_shape=jax.ShapeDtypeStruct((N, D), jnp.float32),
    )(y, alpha, beta)


def _graphnorm_affine(s1, s2, gw, gb, ga, eps=1e-5):
    m = s1 / N
    var = s2 / N - (2.0 * ga - ga * ga) * m * m
    r = lax.rsqrt(var + eps)
    alpha = gw * r
    beta = gb - gw * r * ga * m
    return alpha, beta


def kernel(x, edge_index,
           W0, b0, gw0, gb0, ga0,
           W1, b1, gw1, gb1, ga1,
           W2, b2, gw2, gb2, ga2):
    src = edge_index[0]
    dst = edge_index[1]

    csrc, cloc, counts, deg_flat = _sc_prep(src, dst)
    dinv2d = _dinv(deg_flat.reshape(NPAD, DEG_W))

    params = [(W0, b0, gw0, gb0, ga0),
              (W1, b1, gw1, gb1, ga1),
              (W2, b2, gw2, gb2, ga2)]

    y = x
    alpha = beta = None
    for i, (W, b, gw, gb, ga) in enumerate(params):
        if i == 0:
            Wi = W
            c = jnp.zeros((1, D), jnp.float32)
        else:
            Wi = alpha[0][:, None] * W
            c = beta @ W
        g = _mm(y, Wi, dinv2d, c)
        s_flat = _sc_aggregate(g, csrc, cloc, counts)
        y, s1, s2 = _combine(s_flat.reshape(NPAD, D), g, dinv2d, b[None, :])
        alpha, beta = _graphnorm_affine(s1, s2, gw, gb, ga)
    return _norm(y, alpha, beta)


# speedup vs baseline: 1.2226x; 1.2226x over previous
import functools

import jax
import jax.numpy as jnp
from jax import lax
from jax.experimental import pallas as pl
from jax.experimental.pallas import tpu as pltpu
from jax.experimental.pallas import tpu_sc as plsc

N = 10000
E = 160000
D = 256
BR = 1000

NSC = 2
NT = 16
NW = NSC * NT
RPW = 320
NPAD = NW * RPW
TRASH = RPW
ACC_R = RPW + 1

SUPER = 16000
NSUP = E // SUPER
OSZ = SUPER + 8
STRIDE = NSUP * OSZ + 64
CLEN = NW * 16

RND = 4096
CH = 32
DEG_W = 16

_sc_mesh = plsc.VectorSubcoreMesh(core_axis_name="c", subcore_axis_name="s")


@functools.partial(
    pl.kernel,
    out_type=[
        jax.ShapeDtypeStruct((NW * STRIDE + RND,), jnp.int32),
        jax.ShapeDtypeStruct((NW * STRIDE + RND,), jnp.int32),
        jax.ShapeDtypeStruct((CLEN,), jnp.int32),
        jax.ShapeDtypeStruct((NPAD * DEG_W,), jnp.float32),
    ],
    mesh=_sc_mesh,
    compiler_params=pltpu.CompilerParams(needs_layout_passes=False),
    scratch_types=[
        pltpu.VMEM((SUPER,), jnp.int32),
        pltpu.VMEM((SUPER,), jnp.int32),
        pltpu.VMEM((SUPER,), jnp.int32),
        pltpu.VMEM((SUPER,), jnp.int32),
        pltpu.VMEM((OSZ,), jnp.int32),
        pltpu.VMEM((OSZ,), jnp.int32),
        pltpu.VMEM((ACC_R * DEG_W,), jnp.float32),
        pltpu.VMEM((64,), jnp.int32),
        pltpu.VMEM((64,), jnp.int32),
        pltpu.VMEM((16,), jnp.int32),
        pltpu.SemaphoreType.DMA,
        pltpu.SemaphoreType.DMA,
    ],
)
def _sc_prep(src_hbm, dst_hbm, csrc_hbm, cloc_hbm, cnt_hbm, deg_hbm,
             sA, dA, sB, dB, osrc, oloc, degacc, safes, safel, cbuf,
             semA, semB):
    sc = lax.axis_index("c")
    t = lax.axis_index("s")
    w = sc * NT + t
    wlo = w * RPW
    obase = pl.multiple_of(w * STRIDE, 8)
    iota16 = lax.iota(jnp.int32, 16)
    ones16 = jnp.ones((16,), jnp.float32)

    zero16 = jnp.zeros((16,), jnp.float32)

    def zbody(r, c):
        degacc[pl.ds(r * 16, 16)] = zero16
        return c
    lax.fori_loop(0, ACC_R * DEG_W // 16, zbody, 0)

    for j in range(4):
        safes[pl.ds(j * 16, 16)] = iota16
        safel[pl.ds(j * 16, 16)] = jnp.full((16,), TRASH, jnp.int32)

    sets = ((sA, dA, semA), (sB, dB, semB))

    def issue(sp, st):
        s_v, d_v, sem = st
        base = sp * SUPER
        pltpu.async_copy(src_hbm.at[pl.ds(base, SUPER)], s_v, sem)
        pltpu.async_copy(dst_hbm.at[pl.ds(base, SUPER)], d_v, sem)

    def drain(sp, st):
        s_v, d_v, sem = st
        base = sp * SUPER
        pltpu.make_async_copy(src_hbm.at[pl.ds(base, SUPER)], s_v, sem).wait()
        pltpu.make_async_copy(dst_hbm.at[pl.ds(base, SUPER)], d_v, sem).wait()

    issue(0, sets[0])
    done = jnp.int32(0)
    for sp in range(NSUP):
        cur = sets[sp % 2]
        drain(sp, cur)
        if sp + 1 < NSUP:
            issue(sp + 1, sets[(sp + 1) % 2])
        s_v, d_v, _ = cur

        def vbody(v, tail):
            base = v * 16
            s16 = s_v[pl.ds(base, 16)]
            d16 = d_v[pl.ds(base, 16)]
            loc = d16 - wlo
            owned = (loc >= 0) & (loc < RPW)
            mi = jnp.where(owned, 1, 0).astype(jnp.int32)
            cs = plsc.cumsum(mi)
            idx = tail + cs - 1
            plsc.store_scatter(osrc, [idx], s16, mask=owned)
            plsc.store_scatter(oloc, [idx], loc, mask=owned)
            clamped = jnp.where(owned, loc, TRASH)
            plsc.addupdate_scatter(degacc, [clamped * DEG_W + iota16], ones16)
            return tail + jnp.sum(mi)

        tail = lax.fori_loop(0, SUPER // 16, vbody, jnp.int32(0))
        padl = (-tail) % 8
        pidx = tail + iota16
        pmask = iota16 < padl
        plsc.store_scatter(osrc, [pidx], iota16, mask=pmask)
        plsc.store_scatter(oloc, [pidx], jnp.full((16,), TRASH, jnp.int32),
                           mask=pmask)
        tail = tail + padl
        off = pl.multiple_of(obase + done, 8)
        pltpu.sync_copy(osrc, csrc_hbm.at[pl.ds(off, OSZ)])
        pltpu.sync_copy(oloc, cloc_hbm.at[pl.ds(off, OSZ)])
        done = done + tail

    off = pl.multiple_of(obase + done, 8)
    pltpu.sync_copy(safes, csrc_hbm.at[pl.ds(off, 64)])
    pltpu.sync_copy(safel, cloc_hbm.at[pl.ds(off, 64)])

    cbuf[...] = jnp.broadcast_to(done, (16,)).astype(jnp.int32)
    pltpu.sync_copy(cbuf, cnt_hbm.at[pl.ds(w * 16, 16)])
    pltpu.sync_copy(degacc.at[pl.ds(0, RPW * DEG_W)],
                    deg_hbm.at[pl.ds(wlo * DEG_W, RPW * DEG_W)])


def _bcast_lane(vec16, lane):
    idx = jnp.broadcast_to(lane, (16, 1)).astype(jnp.int32)
    return lax.gather(
        vec16, idx,
        lax.GatherDimensionNumbers(offset_dims=(), collapsed_slice_dims=(0,),
                                   start_index_map=(0,)),
        slice_sizes=(1,),
        mode=lax.GatherScatterMode.PROMISE_IN_BOUNDS)


def _make_aggregate():
    colj = None

    @functools.partial(
        pl.kernel,
        out_type=jax.ShapeDtypeStruct((NPAD * D,), jnp.float32),
        mesh=_sc_mesh,
        compiler_params=pltpu.CompilerParams(needs_layout_passes=False),
        scratch_types=[
            pltpu.VMEM((RND,), jnp.int32),
            pltpu.VMEM((RND,), jnp.int32),
            pltpu.VMEM((CH, D), jnp.float32),
            pltpu.VMEM((CH, D), jnp.float32),
            pltpu.VMEM((ACC_R * D,), jnp.float32),
            pltpu.VMEM((16,), jnp.int32),
            pltpu.SemaphoreType.DMA,
            pltpu.SemaphoreType.DMA,
        ],
    )
    def agg(g_hbm, csrc_hbm, cloc_hbm, cnt_hbm, s_hbm,
            bsrc, bloc, rowA, rowB, acc, cbuf, semA, semB):
        sc = lax.axis_index("c")
        t = lax.axis_index("s")
        w = sc * NT + t
        wlo = w * RPW
        obase = pl.multiple_of(w * STRIDE, 8)
        iota16 = lax.iota(jnp.int32, 16)
        cols = [jnp.int32(j * 16) + iota16 for j in range(D // 16)]
        zero16 = jnp.zeros((16,), jnp.float32)

        def zbody(r, c):
            acc[pl.ds(r * 16, 16)] = zero16
            return c

        lax.fori_loop(0, ACC_R * D // 16, zbody, 0)
        pltpu.sync_copy(cnt_hbm.at[pl.ds(w * 16, 16)], cbuf)
        cnt = jnp.max(cbuf[...])
        nrnd = (cnt + RND - 1) // RND

        def round_body(r, carry):
            rbase = r * RND
            off = pl.multiple_of(obase + rbase, 8)
            pltpu.sync_copy(csrc_hbm.at[pl.ds(off, RND)], bsrc)
            pltpu.sync_copy(cloc_hbm.at[pl.ds(off, RND)], bloc)
            rcnt = jnp.minimum(jnp.int32(RND), cnt - rbase)
            nch = (rcnt + CH - 1) // CH

            def gidx(ch):
                return bsrc.at[pl.ds(ch * CH, CH)]

            def step(ch, rbuf, sem, nbuf, nsem):
                pltpu.make_async_copy(g_hbm.at[gidx(ch)], rbuf, sem).wait()

                @pl.when(ch + 1 < nch)
                def _():
                    pltpu.async_copy(g_hbm.at[gidx(ch + 1)], nbuf, nsem)

                for grp in range(CH // 16):
                    locv = bloc[pl.ds(ch * CH + grp * 16, 16)] * D
                    for l in range(16):
                        locb = jnp.broadcast_to(locv[l], (16,))
                        e = grp * 16 + l
                        vals = [rbuf[e, pl.ds(j * 16, 16)]
                                for j in range(D // 16)]
                        idxs = [locb + cols[j] for j in range(D // 16)]
                        for j in range(D // 16):
                            plsc.addupdate_scatter(acc, [idxs[j]], vals[j])

            @pl.when(nch > 0)
            def _():
                pltpu.async_copy(g_hbm.at[gidx(0)], rowA, semA)

            def chunk_body(ch, c):
                even = (ch % 2) == 0

                @pl.when(even)
                def _():
                    step(ch, rowA, semA, rowB, semB)

                @pl.when(jnp.logical_not(even))
                def _():
                    step(ch, rowB, semB, rowA, semA)

                return c

            lax.fori_loop(0, nch, chunk_body, 0)
            return carry

        lax.fori_loop(0, nrnd, round_body, 0)
        out_base = pl.multiple_of(wlo * D, 8)
        pltpu.sync_copy(acc.at[pl.ds(0, RPW * D)],
                        s_hbm.at[pl.ds(out_base, RPW * D)])

    return agg


_sc_aggregate = _make_aggregate()


def _dinv_body(deg_ref, o_ref):
    deg = 1.0 + jnp.sum(deg_ref[...], axis=1, keepdims=True)
    o_ref[...] = lax.rsqrt(deg)


def _dinv(deg16):
    return pl.pallas_call(
        _dinv_body,
        grid=(N // BR,),
        in_specs=[pl.BlockSpec((BR, DEG_W), lambda i: (i, 0))],
        out_specs=pl.BlockSpec((BR, 1), lambda i: (i, 0)),
        out_shape=jax.ShapeDtypeStruct((N, 1), jnp.float32),
    )(deg16)


def _mm_body(x_ref, w_ref, dinv_ref, c_ref, o_ref):
    o_ref[...] = dinv_ref[...] * (jnp.dot(
        x_ref[...], w_ref[...], preferred_element_type=jnp.float32)
        + c_ref[...])


def _mm(x, w, dinv2d, c):
    grid = (x.shape[0] // BR,)
    return pl.pallas_call(
        _mm_body,
        grid=grid,
        in_specs=[
            pl.BlockSpec((BR, x.shape[1]), lambda i: (i, 0)),
            pl.BlockSpec((x.shape[1], D), lambda i: (0, 0)),
            pl.BlockSpec((BR, 1), lambda i: (i, 0)),
            pl.BlockSpec((1, D), lambda i: (0, 0)),
        ],
        out_specs=pl.BlockSpec((BR, D), lambda i: (i, 0)),
        out_shape=jax.ShapeDtypeStruct((x.shape[0], D), jnp.float32),
    )(x, w, dinv2d, c)


def _combine_body(s_ref, g_ref, dinv_ref, b_ref, y_ref, s1_ref, s2_ref):
    i = pl.program_id(0)
    y = dinv_ref[...] * (s_ref[...] + g_ref[...]) + b_ref[...]
    y_ref[...] = y

    @pl.when(i == 0)
    def _init():
        s1_ref[...] = jnp.zeros_like(s1_ref)
        s2_ref[...] = jnp.zeros_like(s2_ref)

    s1_ref[...] += jnp.sum(y, axis=0, keepdims=True)
    s2_ref[...] += jnp.sum(y * y, axis=0, keepdims=True)


def _combine(s_pad, g, dinv2d, b):
    grid = (N // BR,)
    return pl.pallas_call(
        _combine_body,
        grid=grid,
        in_specs=[
            pl.BlockSpec((BR, D), lambda i: (i, 0)),
            pl.BlockSpec((BR, D), lambda i: (i, 0)),
            pl.BlockSpec((BR, 1), lambda i: (i, 0)),
            pl.BlockSpec((1, D), lambda i: (0, 0)),
        ],
        out_specs=[
            pl.BlockSpec((BR, D), lambda i: (i, 0)),
            pl.BlockSpec((1, D), lambda i: (0, 0)),
            pl.BlockSpec((1, D), lambda i: (0, 0)),
        ],
        out_shape=[
            jax.ShapeDtypeStruct((N, D), jnp.float32),
            jax.ShapeDtypeStruct((1, D), jnp.float32),
            jax.ShapeDtypeStruct((1, D), jnp.float32),
        ],
    )(s_pad, g, dinv2d, b)


def _norm_body(y_ref, al_ref, be_ref, o_ref):
    o_ref[...] = y_ref[...] * al_ref[...] + be_ref[...]


def _norm(y, alpha, beta):
    return pl.pallas_call(
        _norm_body,
        grid=(N // BR,),
        in_specs=[
            pl.BlockSpec((BR, D), lambda i: (i, 0)),
            pl.BlockSpec((1, D), lambda i: (0, 0)),
            pl.BlockSpec((1, D), lambda i: (0, 0)),
        ],
        out_specs=pl.BlockSpec((BR, D), lambda i: (i, 0)),
        out_shape=jax.ShapeDtypeStruct((N, D), jnp.float32),
    )(y, alpha, beta)


def _graphnorm_affine(s1, s2, gw, gb, ga, eps=1e-5):
    m = s1 / N
    var = s2 / N - (2.0 * ga - ga * ga) * m * m
    r = lax.rsqrt(var + eps)
    alpha = gw * r
    beta = gb - gw * r * ga * m
    return alpha, beta


def kernel(x, edge_index,
           W0, b0, gw0, gb0, ga0,
           W1, b1, gw1, gb1, ga1,
           W2, b2, gw2, gb2, ga2):
    src = edge_index[0]
    dst = edge_index[1]

    csrc, cloc, counts, deg_flat = _sc_prep(src, dst)
    dinv2d = _dinv(deg_flat.reshape(NPAD, DEG_W))

    params = [(W0, b0, gw0, gb0, ga0),
              (W1, b1, gw1, gb1, ga1),
              (W2, b2, gw2, gb2, ga2)]

    y = x
    alpha = beta = None
    for i, (W, b, gw, gb, ga) in enumerate(params):
        if i == 0:
            Wi = W
            c = jnp.zeros((1, D), jnp.float32)
        else:
            Wi = alpha[0][:, None] * W
            c = beta @ W
        g = _mm(y, Wi, dinv2d, c)
        s_flat = _sc_aggregate(g, csrc, cloc, counts)
        y, s1, s2 = _combine(s_flat.reshape(NPAD, D), g, dinv2d, b[None, :])
        alpha, beta = _graphnorm_affine(s1, s2, gw, gb, ga)
    return _norm(y, alpha, beta)
